# baseline (speedup 1.0000x reference)
import functools

import numpy as np

import jax
import jax.numpy as jnp
from jax import lax
from jax.experimental import pallas as pl
from jax.experimental.pallas import tpu as pltpu
from jax.experimental.pallas import tpu_sc as plsc

B = 9216
D = 64
V = 1024
NSLICE = 3
BS = B // NSLICE
TOK_BLK = 1024

_NC, _NS = 2, 16
_NW = _NC * _NS
_BPW = BS // _NW


def _dist_argmax_body(x_ref, e_ref, idx_ref):
    x = x_ref[...]
    e = e_ref[...]
    xn = x / jnp.maximum(jnp.sqrt(jnp.sum(x * x, axis=-1, keepdims=True)), 1e-12)
    en = e / jnp.maximum(jnp.sqrt(jnp.sum(e * e, axis=-1, keepdims=True)), 1e-12)
    dist = lax.dot_general(xn, en, (((1,), (1,)), ((), ())),
                           preferred_element_type=jnp.float32)
    m = jnp.max(dist, axis=-1, keepdims=True)
    desc = jnp.float32(V) - lax.broadcasted_iota(
        jnp.int32, dist.shape, 1).astype(jnp.float32)
    cand = jnp.where(dist == m, desc, jnp.float32(0.0))
    winner = jnp.max(cand, axis=-1)
    idx = (jnp.float32(V) - winner).astype(jnp.int32)
    idx_ref[...] = jnp.clip(idx, 0, V - 1)


def _tc_argmax(x_slice, embed):
    grid = BS // TOK_BLK
    return pl.pallas_call(
        _dist_argmax_body,
        grid=(grid,),
        in_specs=[
            pl.BlockSpec((TOK_BLK, D), lambda i: (i, 0)),
            pl.BlockSpec((V, D), lambda i: (0, 0)),
        ],
        out_specs=pl.BlockSpec((TOK_BLK,), lambda i: (i,)),
        out_shape=jax.ShapeDtypeStruct((BS,), jnp.int32),
    )(x_slice, embed)


@functools.cache
def _make_sc_gather():
    @functools.partial(
        pl.kernel,
        mesh=plsc.VectorSubcoreMesh(core_axis_name="c", subcore_axis_name="s"),
        out_type=jax.ShapeDtypeStruct((BS, D), jnp.float32),
        scratch_types=[
            pltpu.VMEM((_BPW,), jnp.int32),
            pltpu.VMEM((_BPW, D), jnp.float32),
            pltpu.SemaphoreType.DMA,
        ],
        compiler_params=pltpu.CompilerParams(use_tc_tiling_on_sc=False),
    )
    def _sc_gather(table_hbm, idx_hbm, out_hbm, idx_v, rows_v, sem):
        wid = lax.axis_index("s") * _NC + lax.axis_index("c")
        base = wid * _BPW
        pltpu.sync_copy(idx_hbm.at[pl.ds(base, _BPW)], idx_v)
        pltpu.async_copy(table_hbm.at[idx_v], rows_v, sem).wait()
        pltpu.sync_copy(rows_v, out_hbm.at[pl.ds(base, _BPW)])

    return _sc_gather


def kernel(x, embed):
    shape = x.shape
    x_flat = x.reshape(-1, shape[-1])
    sc = _make_sc_gather()
    idxs = [_tc_argmax(x_flat[i * BS:(i + 1) * BS], embed)
            for i in range(NSLICE)]
    quants = [sc(embed, idxs[i]) for i in range(NSLICE)]
    idx = jnp.concatenate(idxs)
    quant = jnp.concatenate(quants)
    return quant.reshape(shape), idx.reshape(shape[:-1])


# speedup vs baseline: 1.1645x; 1.1645x over previous
import functools

import numpy as np

import jax
import jax.numpy as jnp
from jax import lax
from jax.experimental import pallas as pl
from jax.experimental.pallas import tpu as pltpu
from jax.experimental.pallas import tpu_sc as plsc

B = 9216
D = 64
V = 1024
TOK_BLK = 1024

_NC, _NS = 2, 16
_NW = _NC * _NS
_BPW = B // _NW


def _dist_argmax_body(x_ref, e_ref, idx_ref):
    x = x_ref[...]
    e = e_ref[...]
    xn = x / jnp.maximum(jnp.sqrt(jnp.sum(x * x, axis=-1, keepdims=True)), 1e-12)
    en = e / jnp.maximum(jnp.sqrt(jnp.sum(e * e, axis=-1, keepdims=True)), 1e-12)
    dist = lax.dot_general(xn, en, (((1,), (1,)), ((), ())),
                           preferred_element_type=jnp.float32)
    m = jnp.max(dist, axis=-1, keepdims=True)
    desc = jnp.float32(V) - lax.broadcasted_iota(
        jnp.int32, dist.shape, 1).astype(jnp.float32)
    cand = jnp.where(dist == m, desc, jnp.float32(0.0))
    winner = jnp.max(cand, axis=-1)
    idx = (jnp.float32(V) - winner).astype(jnp.int32)
    idx_ref[...] = jnp.clip(idx, 0, V - 1)


def _tc_argmax(x_flat, embed):
    grid = B // TOK_BLK
    return pl.pallas_call(
        _dist_argmax_body,
        grid=(grid,),
        in_specs=[
            pl.BlockSpec((TOK_BLK, D), lambda i: (i, 0)),
            pl.BlockSpec((V, D), lambda i: (0, 0)),
        ],
        out_specs=pl.BlockSpec((TOK_BLK,), lambda i: (i,)),
        out_shape=jax.ShapeDtypeStruct((B,), jnp.int32),
    )(x_flat, embed)


_CHUNK = 96
_NCHUNK = _BPW // _CHUNK
_DP = 128


@functools.cache
def _make_sc_gather():
    @functools.partial(
        pl.kernel,
        mesh=plsc.VectorSubcoreMesh(core_axis_name="c", subcore_axis_name="s"),
        out_type=jax.ShapeDtypeStruct((B, D), jnp.float32),
        scratch_types=[
            pltpu.VMEM((_BPW,), jnp.int32),
            pltpu.VMEM((_BPW, D), jnp.float32),
            pltpu.SemaphoreType.DMA,
        ],
        compiler_params=pltpu.CompilerParams(use_tc_tiling_on_sc=False),
    )
    def _sc_gather(table_hbm, idx_hbm, out_hbm, idx_v, rows_v, sem):
        wid = lax.axis_index("s") * _NC + lax.axis_index("c")
        base = wid * _BPW
        pltpu.sync_copy(idx_hbm.at[pl.ds(base, _BPW)], idx_v)
        copies = [
            pltpu.async_copy(table_hbm.at[idx_v.at[pl.ds(c * _CHUNK, _CHUNK)]],
                             rows_v.at[pl.ds(c * _CHUNK, _CHUNK)], sem)
            for c in range(_NCHUNK)
        ]
        for cp in copies:
            cp.wait()
        pltpu.sync_copy(rows_v, out_hbm.at[pl.ds(base, _BPW)])

    return _sc_gather


def kernel(x, embed):
    shape = x.shape
    x_flat = x.reshape(-1, shape[-1])
    idx = _tc_argmax(x_flat, embed)
    quant = _make_sc_gather()(embed, idx)
    return quant.reshape(shape), idx.reshape(shape[:-1])


# speedup vs baseline: 1.4136x; 1.2139x over previous
import functools

import numpy as np

import jax
import jax.numpy as jnp
from jax import lax
from jax.experimental import pallas as pl
from jax.experimental.pallas import tpu as pltpu
from jax.experimental.pallas import tpu_sc as plsc

B = 9216
D = 64
V = 1024
TOK_BLK = 3072

_NC, _NS = 2, 16
_NW = _NC * _NS
_BPW = B // _NW


def _dist_argmax_body(x_ref, e_ref, idx_ref):
    x = x_ref[...]
    e = e_ref[...]
    xn = x / jnp.maximum(jnp.sqrt(jnp.sum(x * x, axis=-1, keepdims=True)), 1e-12)
    en = e / jnp.maximum(jnp.sqrt(jnp.sum(e * e, axis=-1, keepdims=True)), 1e-12)
    dist = lax.dot_general(en, xn, (((1,), (1,)), ((), ())),
                           preferred_element_type=jnp.float32)
    m = jnp.max(dist, axis=0, keepdims=True)
    desc = jnp.float32(V) - lax.broadcasted_iota(
        jnp.int32, dist.shape, 0).astype(jnp.float32)
    cand = jnp.where(dist == m, desc, jnp.float32(0.0))
    winner = jnp.max(cand, axis=0)
    idx = (jnp.float32(V) - winner).astype(jnp.int32)
    idx_ref[...] = jnp.clip(idx, 0, V - 1)


def _tc_argmax(x_flat, embed):
    grid = B // TOK_BLK
    return pl.pallas_call(
        _dist_argmax_body,
        grid=(grid,),
        in_specs=[
            pl.BlockSpec((TOK_BLK, D), lambda i: (i, 0)),
            pl.BlockSpec((V, D), lambda i: (0, 0)),
        ],
        out_specs=pl.BlockSpec((TOK_BLK,), lambda i: (i,)),
        out_shape=jax.ShapeDtypeStruct((B,), jnp.int32),
    )(x_flat, embed)


_CHUNK = 96
_NCHUNK = _BPW // _CHUNK
_DP = 128


@functools.cache
def _make_sc_gather():
    @functools.partial(
        pl.kernel,
        mesh=plsc.VectorSubcoreMesh(core_axis_name="c", subcore_axis_name="s"),
        out_type=jax.ShapeDtypeStruct((B, D), jnp.float32),
        scratch_types=[
            pltpu.VMEM((_BPW,), jnp.int32),
            pltpu.VMEM((_BPW, D), jnp.float32),
            pltpu.SemaphoreType.DMA,
        ],
        compiler_params=pltpu.CompilerParams(use_tc_tiling_on_sc=False),
    )
    def _sc_gather(table_hbm, idx_hbm, out_hbm, idx_v, rows_v, sem):
        wid = lax.axis_index("s") * _NC + lax.axis_index("c")
        base = wid * _BPW
        pltpu.sync_copy(idx_hbm.at[pl.ds(base, _BPW)], idx_v)
        copies = [
            pltpu.async_copy(table_hbm.at[idx_v.at[pl.ds(c * _CHUNK, _CHUNK)]],
                             rows_v.at[pl.ds(c * _CHUNK, _CHUNK)], sem)
            for c in range(_NCHUNK)
        ]
        for cp in copies:
            cp.wait()
        pltpu.sync_copy(rows_v, out_hbm.at[pl.ds(base, _BPW)])

    return _sc_gather


def kernel(x, embed):
    shape = x.shape
    x_flat = x.reshape(-1, shape[-1])
    idx = _tc_argmax(x_flat, embed)
    quant = _make_sc_gather()(embed, idx)
    return quant.reshape(shape), idx.reshape(shape[:-1])
